# Initial kernel scaffold; baseline (speedup 1.0000x reference)
import functools

import jax
import jax.numpy as jnp
from jax import lax
from jax.experimental import pallas as pl
from jax.experimental.pallas import tpu as pltpu
from jax.experimental.pallas import tpu_sc as plsc

N = 10000
D = 128
NC = 2
NS = 16
NW = NC * NS
CHUNK = 128
LANES = 16
ZROWS = 125
ROWS_PER_TILE = N // NS


def _sc_aggregate(x, src, dst, w, epw):
    nchunk = epw // CHUNK
    mesh = plsc.VectorSubcoreMesh(
        core_axis_name="c", subcore_axis_name="s",
        num_cores=NC, num_subcores=NS)

    @functools.partial(
        pl.kernel,
        out_type=jax.ShapeDtypeStruct((NC * N, D), jnp.float32),
        mesh=mesh,
        scratch_types=[
            pltpu.VMEM((CHUNK,), jnp.int32),
            pltpu.VMEM((CHUNK,), jnp.int32),
            pltpu.VMEM((CHUNK,), jnp.float32),
            pltpu.VMEM((CHUNK, D), jnp.float32),
            pltpu.VMEM((ZROWS, D), jnp.float32),
            pltpu.VMEM_SHARED((N, D), jnp.float32),
            pltpu.SemaphoreType.DMA,
        ],
    )
    def agg(x_hbm, src_hbm, dst_hbm, w_hbm, out_hbm,
            src_v, dst_v, w_v, gath_v, zbuf, accum, sem):
        cid = lax.axis_index("c")
        sid = lax.axis_index("s")
        wid = sid * NC + cid

        zero16 = jnp.zeros((LANES,), jnp.float32)

        @pl.loop(0, ZROWS)
        def _(r):
            for c in range(D // LANES):
                zbuf[r, pl.ds(c * LANES, LANES)] = zero16

        @pl.loop(0, ROWS_PER_TILE // ZROWS)
        def _(k):
            pltpu.sync_copy(
                zbuf, accum.at[pl.ds(sid * ROWS_PER_TILE + k * ZROWS, ZROWS)])

        plsc.subcore_barrier()

        base = wid * epw

        @pl.loop(0, nchunk)
        def _(j):
            off = base + j * CHUNK
            pltpu.sync_copy(src_hbm.at[pl.ds(off, CHUNK)], src_v)
            pltpu.sync_copy(dst_hbm.at[pl.ds(off, CHUNK)], dst_v)
            pltpu.sync_copy(w_hbm.at[pl.ds(off, CHUNK)], w_v)
            pltpu.async_copy(x_hbm.at[src_v], gath_v, sem).wait()

            @pl.loop(0, CHUNK)
            def _(i):
                ws = jnp.full((LANES,), w_v[i], jnp.float32)
                for c in range(D // LANES):
                    sl = pl.ds(c * LANES, LANES)
                    gath_v[i, sl] = gath_v[i, sl] * ws

            pltpu.sync_copy(gath_v, accum.at[dst_v], add=True)

        plsc.subcore_barrier()

        row0 = sid * ROWS_PER_TILE
        pltpu.sync_copy(accum.at[pl.ds(row0, ROWS_PER_TILE)],
                        out_hbm.at[pl.ds(cid * N + row0, ROWS_PER_TILE)])

    return agg(x, src, dst, w)


def _tc_finish(p0, p1, W):
    BR = 2000

    def body(p0_ref, p1_ref, w_ref, o_ref):
        pre = p0_ref[...] + p1_ref[...]
        o_ref[...] = lax.dot_general(
            pre, w_ref[...], (((1,), (1,)), ((), ())),
            preferred_element_type=jnp.float32)

    return pl.pallas_call(
        body,
        grid=(N // BR,),
        in_specs=[
            pl.BlockSpec((BR, D), lambda i: (i, 0)),
            pl.BlockSpec((BR, D), lambda i: (i, 0)),
            pl.BlockSpec((D, D), lambda i: (0, 0)),
        ],
        out_specs=pl.BlockSpec((BR, D), lambda i: (i, 0)),
        out_shape=jax.ShapeDtypeStruct((N, D), jnp.float32),
    )(p0, p1, W)


def kernel(ego_embeddings, edge_index, edge_weight, W):
    E = edge_weight.shape[0]
    src = edge_index[0].astype(jnp.int32)
    dst = edge_index[1].astype(jnp.int32)
    w = edge_weight.astype(jnp.float32)

    epw = -(-E // NW)
    epw = -(-epw // CHUNK) * CHUNK
    pad = epw * NW - E
    if pad:
        src = jnp.concatenate([src, jnp.zeros((pad,), jnp.int32)])
        dst = jnp.concatenate([dst, jnp.zeros((pad,), jnp.int32)])
        w = jnp.concatenate([w, jnp.zeros((pad,), jnp.float32)])

    partials = _sc_aggregate(ego_embeddings, src, dst, w, epw)
    return _tc_finish(partials[:N], partials[N:], W)


# speedup vs baseline: 3.3744x; 3.3744x over previous
import functools

import jax
import jax.numpy as jnp
from jax import lax
from jax.experimental import pallas as pl
from jax.experimental.pallas import tpu as pltpu
from jax.experimental.pallas import tpu_sc as plsc

N = 10000
NPAD = 10240
D = 128
NC = 2
NS = 16
NW = NC * NS
CHUNK = 128
LANES = 16
ZROWS = 128
ROWS_PER_TILE = NPAD // NS


def _sc_aggregate(x, src, dst, w, epw):
    nchunk = epw // CHUNK
    mesh = plsc.VectorSubcoreMesh(
        core_axis_name="c", subcore_axis_name="s",
        num_cores=NC, num_subcores=NS)

    @functools.partial(
        pl.kernel,
        out_type=jax.ShapeDtypeStruct((NC * NPAD, D), jnp.float32),
        mesh=mesh,
        scratch_types=[
            pltpu.VMEM((CHUNK,), jnp.int32),
            pltpu.VMEM((CHUNK,), jnp.int32),
            pltpu.VMEM((CHUNK,), jnp.float32),
            pltpu.VMEM((CHUNK, D), jnp.float32),
            pltpu.VMEM((ZROWS, D), jnp.float32),
            pltpu.VMEM_SHARED((NPAD, D), jnp.float32),
            pltpu.SemaphoreType.DMA,
        ],
    )
    def agg(x_hbm, src_hbm, dst_hbm, w_hbm, out_hbm,
            src_v, dst_v, w_v, gath_v, zbuf, accum, sem):
        cid = lax.axis_index("c")
        sid = lax.axis_index("s")
        wid = sid * NC + cid

        zero16 = jnp.zeros((LANES,), jnp.float32)

        @pl.loop(0, ZROWS)
        def _(r):
            for c in range(D // LANES):
                zbuf[r, pl.ds(c * LANES, LANES)] = zero16

        @pl.loop(0, ROWS_PER_TILE // ZROWS)
        def _(k):
            pltpu.sync_copy(
                zbuf, accum.at[pl.ds(sid * ROWS_PER_TILE + k * ZROWS, ZROWS)])

        plsc.subcore_barrier()

        base = wid * epw

        @pl.loop(0, nchunk)
        def _(j):
            off = base + j * CHUNK
            pltpu.sync_copy(src_hbm.at[pl.ds(off, CHUNK)], src_v)
            pltpu.sync_copy(dst_hbm.at[pl.ds(off, CHUNK)], dst_v)
            pltpu.sync_copy(w_hbm.at[pl.ds(off, CHUNK)], w_v)
            pltpu.async_copy(x_hbm.at[src_v], gath_v, sem).wait()

            @pl.loop(0, CHUNK // LANES)
            def _(g):
                wv = w_v[pl.ds(g * LANES, LANES)]
                for l in range(LANES):
                    ws = jnp.full((LANES,), wv[l], jnp.float32)
                    row = g * LANES + l
                    for c in range(D // LANES):
                        sl = pl.ds(c * LANES, LANES)
                        gath_v[row, sl] = gath_v[row, sl] * ws

            pltpu.sync_copy(gath_v, accum.at[dst_v], add=True)

        plsc.subcore_barrier()

        row0 = sid * ROWS_PER_TILE
        pltpu.sync_copy(accum.at[pl.ds(row0, ROWS_PER_TILE)],
                        out_hbm.at[pl.ds(cid * NPAD + row0, ROWS_PER_TILE)])

    return agg(x, src, dst, w)


def _tc_finish(p0, p1, W):
    BR = 2000

    def body(p0_ref, p1_ref, w_ref, o_ref):
        pre = p0_ref[...] + p1_ref[...]
        o_ref[...] = lax.dot_general(
            pre, w_ref[...], (((1,), (1,)), ((), ())),
            preferred_element_type=jnp.float32)

    return pl.pallas_call(
        body,
        grid=(N // BR,),
        in_specs=[
            pl.BlockSpec((BR, D), lambda i: (i, 0)),
            pl.BlockSpec((BR, D), lambda i: (i, 0)),
            pl.BlockSpec((D, D), lambda i: (0, 0)),
        ],
        out_specs=pl.BlockSpec((BR, D), lambda i: (i, 0)),
        out_shape=jax.ShapeDtypeStruct((N, D), jnp.float32),
    )(p0, p1, W)


def kernel(ego_embeddings, edge_index, edge_weight, W):
    E = edge_weight.shape[0]
    src = edge_index[0].astype(jnp.int32)
    dst = edge_index[1].astype(jnp.int32)
    w = edge_weight.astype(jnp.float32)

    epw = -(-E // NW)
    epw = -(-epw // CHUNK) * CHUNK
    pad = epw * NW - E
    if pad:
        src = jnp.concatenate([src, jnp.zeros((pad,), jnp.int32)])
        dst = jnp.concatenate([dst, jnp.zeros((pad,), jnp.int32)])
        w = jnp.concatenate([w, jnp.zeros((pad,), jnp.float32)])

    partials = _sc_aggregate(ego_embeddings, src, dst, w, epw)
    return _tc_finish(partials[:N], partials[NPAD:NPAD + N], W)
